# baseline (speedup 1.0000x reference)
import functools

import jax
import jax.numpy as jnp
from jax import lax
from jax.experimental import pallas as pl
from jax.experimental.pallas import tpu as pltpu
from jax.experimental.pallas import tpu_sc as plsc

N = 100000
E = 1600000
IN_DIM = 2
OUT_DIM = 214

NPAD = 100352
NB128 = NPAD // 128
TRASH = NPAD - 1

NC = 2
NSUB = 16
NW = NC * NSUB
GSZ = 2048
NB = 5
G_A = 15
G_B = 35
EPAD = NSUB * (G_A + G_B) * GSZ
NSLICE = NPAD // NSUB

RB = 512
GRID = NPAD // RB

@functools.cache
def _mesh():
    return plsc.VectorSubcoreMesh(core_axis_name="c", subcore_axis_name="s",
                                  num_cores=NC, num_subcores=NSUB)


def _deg_body(dst_hbm, zeros_hbm, out_hbm, *scr):
    idxs = scr[0:NB]
    ones_v, shared_deg, ssem = scr[NB:]
    c = lax.axis_index("c")
    t = lax.axis_index("s")
    gc = jnp.where(c == 0, G_A, G_B)
    base = (c * NSUB * G_A + t * gc) * GSZ

    def fill(i, carry):
        ones_v[pl.ds(i * 16, 16)] = jnp.ones((16,), jnp.float32)
        return carry

    lax.fori_loop(0, GSZ // 16, fill, 0)

    @pl.when(t == 0)
    def _():
        pltpu.sync_copy(zeros_hbm, shared_deg)

    plsc.subcore_barrier()

    def body(i, carry):
        g0 = i * NB
        sd = []
        for j in range(NB):
            pltpu.sync_copy(dst_hbm.at[pl.ds(base + (g0 + j) * GSZ, GSZ)],
                            idxs[j])
            sd.append(pltpu.async_copy(ones_v, shared_deg.at[idxs[j]], ssem,
                                       add=True))
        for d in sd:
            d.wait()
        return carry

    lax.fori_loop(0, gc // NB, body, 0)
    plsc.subcore_barrier()
    pltpu.sync_copy(shared_deg.at[pl.ds(t * NSLICE, NSLICE)],
                    out_hbm.at[pl.ds(c * NPAD + t * NSLICE, NSLICE)])


@functools.cache
def _deg_call():
    return pl.kernel(
        _deg_body,
        out_type=jax.ShapeDtypeStruct((NC * NPAD,), jnp.float32),
        mesh=_mesh(),
        scratch_types=(
            [pltpu.VMEM((GSZ,), jnp.int32)] * NB
            + [pltpu.VMEM((GSZ,), jnp.float32),
               pltpu.VMEM_SHARED((NPAD,), jnp.float32),
               pltpu.SemaphoreType.DMA]
        ),
    )


def _sct_body(src_hbm, dst_hbm, y0_hbm, y1_hbm, out0_hbm, out1_hbm, *scr):
    sidxs = scr[0:NB]
    didxs = scr[NB:2 * NB]
    vals0 = scr[2 * NB:3 * NB]
    vals1 = scr[3 * NB:4 * NB]
    gsems = scr[4 * NB:5 * NB]
    sh_y0, sh_y1, sh_s0, sh_s1, ssa, ssb = scr[5 * NB:]
    c = lax.axis_index("c")
    t = lax.axis_index("s")
    gc = jnp.where(c == 0, G_A, G_B)
    base = (c * NSUB * G_A + t * gc) * GSZ
    lo = t * NSLICE

    zb = vals0[0]

    def zfill(k, carry):
        zb[pl.ds(k * 16, 16)] = jnp.zeros((16,), jnp.float32)
        return carry

    lax.fori_loop(0, GSZ // 16, zfill, 0)
    for shs in (sh_s0, sh_s1):
        def zslice(k, carry, shs=shs):
            pltpu.sync_copy(zb, shs.at[pl.ds(lo + k * GSZ, GSZ)])
            return carry

        lax.fori_loop(0, NSLICE // GSZ, zslice, 0)
        pltpu.sync_copy(zb.at[pl.ds(0, NSLICE % GSZ)],
                        shs.at[pl.ds(lo + (NSLICE // GSZ) * GSZ,
                                     NSLICE % GSZ)])
    pltpu.sync_copy(y0_hbm.at[pl.ds(lo, NSLICE)], sh_y0.at[pl.ds(lo, NSLICE)])
    pltpu.sync_copy(y1_hbm.at[pl.ds(lo, NSLICE)], sh_y1.at[pl.ds(lo, NSLICE)])
    plsc.subcore_barrier()

    def sat(g):
        return src_hbm.at[pl.ds(base + g * GSZ, GSZ)]

    def dat(g):
        return dst_hbm.at[pl.ds(base + g * GSZ, GSZ)]

    def body(i, carry):
        g0 = i * NB
        gd = []
        for j in range(NB):
            pltpu.sync_copy(sat(g0 + j), sidxs[j])
            pltpu.sync_copy(dat(g0 + j), didxs[j])
            gd.append((pltpu.async_copy(sh_y0.at[sidxs[j]], vals0[j],
                                        gsems[j]),
                       pltpu.async_copy(sh_y1.at[sidxs[j]], vals1[j],
                                        gsems[j])))
        sd = []
        for j in range(NB):
            gd[j][0].wait()
            gd[j][1].wait()
            sd.append((pltpu.async_copy(vals0[j], sh_s0.at[didxs[j]], ssa,
                                        add=True),
                       pltpu.async_copy(vals1[j], sh_s1.at[didxs[j]], ssb,
                                        add=True)))
        for j in range(NB):
            sd[j][0].wait()
            sd[j][1].wait()
        return carry

    lax.fori_loop(0, gc // NB, body, 0)
    plsc.subcore_barrier()
    pltpu.sync_copy(sh_s0.at[pl.ds(lo, NSLICE)],
                    out0_hbm.at[pl.ds(c * NPAD + lo, NSLICE)])
    pltpu.sync_copy(sh_s1.at[pl.ds(lo, NSLICE)],
                    out1_hbm.at[pl.ds(c * NPAD + lo, NSLICE)])


@functools.cache
def _sct_call():
    return pl.kernel(
        _sct_body,
        out_type=(
            jax.ShapeDtypeStruct((NC * NPAD,), jnp.float32),
            jax.ShapeDtypeStruct((NC * NPAD,), jnp.float32),
        ),
        mesh=_mesh(),
        scratch_types=(
            [pltpu.VMEM((GSZ,), jnp.int32)] * (2 * NB)
            + [pltpu.VMEM((GSZ,), jnp.float32)] * (2 * NB)
            + [pltpu.SemaphoreType.DMA] * NB
            + [pltpu.VMEM_SHARED((NPAD,), jnp.float32)] * 4
            + [pltpu.SemaphoreType.DMA] * 2
        ),
    )


def _prep_body(degp_ref, xT_ref, dinv_ref, yT_ref):
    deg = degp_ref[0] + degp_ref[1] + 1.0
    dinv = lax.rsqrt(deg)
    dinv_ref[...] = dinv
    yT_ref[0] = xT_ref[0] * dinv
    yT_ref[1] = xT_ref[1] * dinv


def _prep_call(degp3, xT3):
    return pl.pallas_call(
        _prep_body,
        out_shape=(
            jax.ShapeDtypeStruct((NB128, 128), jnp.float32),
            jax.ShapeDtypeStruct((IN_DIM, NB128, 128), jnp.float32),
        ),
    )(degp3, xT3)


D1R = 8
D1G = NB128 // D1R


def _d1_body(o00, o01, y0b, o10, o11, y1b, dv, WTr, bc, gc, betac,
             acc0_ref, acc1_ref, scale_ref, shift_ref, ssum, ssq):
    i = pl.program_id(0)

    @pl.when(i == 0)
    def _():
        ssum[...] = jnp.zeros_like(ssum)
        ssq[...] = jnp.zeros_like(ssq)

    acc0 = (o00[...] + o01[...] + y0b[...]) * dv[...]
    acc1 = (o10[...] + o11[...] + y1b[...]) * dv[...]
    acc0_ref[...] = acc0
    acc1_ref[...] = acc1
    w0 = WTr[:, 0:1]
    w1 = WTr[:, 1:2]
    lane = lax.broadcasted_iota(jnp.int32, (1, 128), 1)
    su = ssum[...]
    sq = ssq[...]
    for j in range(D1R):
        h1 = jnp.maximum(w0 * acc0[j:j + 1, :] + w1 * acc1[j:j + 1, :]
                         + bc[...], 0.0)
        node = (i * D1R + j) * 128 + lane
        h1 = jnp.where(node < N, h1, 0.0)
        su = su + h1
        sq = sq + h1 * h1
    ssum[...] = su
    ssq[...] = sq

    @pl.when(i == pl.num_programs(0) - 1)
    def _():
        mean = jnp.sum(su, axis=1, keepdims=True) * (1.0 / N)
        ex2 = jnp.sum(sq, axis=1, keepdims=True) * (1.0 / N)
        var = ex2 - mean * mean
        rstd = lax.rsqrt(var + 1e-5)
        scale_ref[...] = gc[...] * rstd
        shift_ref[...] = betac[...] - mean * gc[...] * rstd


def _d1_call(o00, o01, y0b, o10, o11, y1b, dinvb, WT, bc, gc, betac):
    blk = pl.BlockSpec((D1R, 128), lambda i: (i, 0))
    col = pl.BlockSpec((OUT_DIM, 1), lambda i: (0, 0))
    return pl.pallas_call(
        _d1_body,
        grid=(D1G,),
        in_specs=[blk, blk, blk, blk, blk, blk, blk,
                  pl.BlockSpec((OUT_DIM, IN_DIM), lambda i: (0, 0)),
                  col, col, col],
        out_specs=(blk, blk, col, col),
        out_shape=(
            jax.ShapeDtypeStruct((NB128, 128), jnp.float32),
            jax.ShapeDtypeStruct((NB128, 128), jnp.float32),
            jax.ShapeDtypeStruct((OUT_DIM, 1), jnp.float32),
            jax.ShapeDtypeStruct((OUT_DIM, 1), jnp.float32),
        ),
        scratch_shapes=[
            pltpu.VMEM((OUT_DIM, 128), jnp.float32),
            pltpu.VMEM((OUT_DIM, 128), jnp.float32),
        ],
    )(o00, o01, y0b, o10, o11, y1b, dinvb, WT, bc, gc, betac)


D2R = 8
D2G = NB128 // D2R


def _d2_body(acc0b, acc1b, WTr, bc, scale, shift, fcW_r, fcb_r,
             out1_ref, out2_ref):
    w0 = WTr[:, 0:1]
    w1 = WTr[:, 1:2]
    sc = scale[...]
    sh = shift[...]
    ones = jnp.ones((OUT_DIM, 1), jnp.float32)
    for j in range(D2R):
        h1 = jnp.maximum(w0 * acc0b[j:j + 1, :] + w1 * acc1b[j:j + 1, :]
                         + bc[...], 0.0)
        hbnT = h1 * sc + sh
        h2 = lax.dot_general(hbnT.astype(jnp.bfloat16), fcW_r[...],
                             (((0,), (0,)), ((), ())),
                             preferred_element_type=jnp.float32)
        h2 = h2 + fcb_r[...]
        s = lax.dot_general(jnp.exp(h2), ones,
                            (((1,), (0,)), ((), ())),
                            preferred_element_type=jnp.float32)
        lse = jnp.log(s)
        out1_ref[pl.ds(j * 128, 128), :] = h2 - lse
        out2_ref[pl.ds(j * 128, 128), :] = h2


def _d2_call(acc0, acc1, WT, bc, scale, shift, fcW, fcb2):
    blk = pl.BlockSpec((D2R, 128), lambda i: (i, 0))
    col = pl.BlockSpec((OUT_DIM, 1), lambda i: (0, 0))
    return pl.pallas_call(
        _d2_body,
        grid=(D2G,),
        in_specs=[blk, blk,
                  pl.BlockSpec((OUT_DIM, IN_DIM), lambda i: (0, 0)),
                  col, col, col,
                  pl.BlockSpec((OUT_DIM, OUT_DIM), lambda i: (0, 0)),
                  pl.BlockSpec((1, OUT_DIM), lambda i: (0, 0))],
        out_specs=(
            pl.BlockSpec((D2R * 128, OUT_DIM), lambda i: (i, 0)),
            pl.BlockSpec((D2R * 128, OUT_DIM), lambda i: (i, 0)),
        ),
        out_shape=(
            jax.ShapeDtypeStruct((N, OUT_DIM), jnp.float32),
            jax.ShapeDtypeStruct((N, OUT_DIM), jnp.float32),
        ),
    )(acc0, acc1, WT, bc, scale, shift, fcW, fcb2)


def kernel(x, edge_index, W, b, gamma, beta, fcW, fcb):
    ei = edge_index.astype(jnp.int32)
    padlen = EPAD - E
    pad = jnp.full((padlen,), TRASH, jnp.int32)
    src3 = jnp.concatenate([ei[0], pad])
    dst3 = jnp.concatenate([ei[1], pad])

    xpad = jnp.pad(x, ((0, NPAD - N), (0, 0)))
    xT3 = xpad.T.reshape(IN_DIM, NB128, 128)
    zeros_n = jnp.zeros((NPAD,), jnp.float32)

    degp = _deg_call()(dst3, zeros_n).reshape(NC, NB128, 128)
    dinvb, yT = _prep_call(degp, xT3)
    y0 = yT[0].reshape(NPAD)
    y1 = yT[1].reshape(NPAD)

    s0f, s1f = _sct_call()(src3, dst3, y0, y1)
    s0p = s0f.reshape(NC, NB128, 128)
    s1p = s1f.reshape(NC, NB128, 128)

    WT = W.T
    bc = b.reshape(OUT_DIM, 1)
    gc = gamma.reshape(OUT_DIM, 1)
    betac = beta.reshape(OUT_DIM, 1)
    fcb2 = fcb.reshape(1, OUT_DIM)
    acc0, acc1, scale, shift = _d1_call(
        s0p[0], s0p[1], yT[0], s1p[0], s1p[1], yT[1],
        dinvb, WT, bc, gc, betac)
    out1, out2 = _d2_call(acc0, acc1, WT, bc, scale, shift,
                          fcW.astype(jnp.bfloat16), fcb2)
    return out1, out2


# speedup vs baseline: 1.0905x; 1.0905x over previous
import functools

import jax
import jax.numpy as jnp
from jax import lax
from jax.experimental import pallas as pl
from jax.experimental.pallas import tpu as pltpu
from jax.experimental.pallas import tpu_sc as plsc

N = 100000
E = 1600000
IN_DIM = 2
OUT_DIM = 214

NPAD = 100352
NB128 = NPAD // 128
TRASH = NPAD - 1

NC = 2
NSUB = 16
NW = NC * NSUB
GSZ = 2048
NB = 5
G_A = 35
G_B = 15
EPAD = NSUB * (G_A + G_B) * GSZ
NSLICE = NPAD // NSUB

RB = 512
GRID = NPAD // RB

@functools.cache
def _mesh():
    return plsc.VectorSubcoreMesh(core_axis_name="c", subcore_axis_name="s",
                                  num_cores=NC, num_subcores=NSUB)


def _deg_body(dst_hbm, zeros_hbm, out_hbm, *scr):
    idxs = scr[0:NB]
    ones_v, shared_deg, ssem = scr[NB:]
    c = lax.axis_index("c")
    t = lax.axis_index("s")
    gc = jnp.where(c == 0, G_A, G_B)
    base = (c * NSUB * G_A + t * gc) * GSZ

    def fill(i, carry):
        ones_v[pl.ds(i * 16, 16)] = jnp.ones((16,), jnp.float32)
        return carry

    lax.fori_loop(0, GSZ // 16, fill, 0)

    @pl.when(t == 0)
    def _():
        pltpu.sync_copy(zeros_hbm, shared_deg)

    plsc.subcore_barrier()

    def body(i, carry):
        g0 = i * NB
        sd = []
        for j in range(NB):
            pltpu.sync_copy(dst_hbm.at[pl.ds(base + (g0 + j) * GSZ, GSZ)],
                            idxs[j])
            sd.append(pltpu.async_copy(ones_v, shared_deg.at[idxs[j]], ssem,
                                       add=True))
        for d in sd:
            d.wait()
        return carry

    lax.fori_loop(0, gc // NB, body, 0)
    plsc.subcore_barrier()
    pltpu.sync_copy(shared_deg.at[pl.ds(t * NSLICE, NSLICE)],
                    out_hbm.at[pl.ds(c * NPAD + t * NSLICE, NSLICE)])


@functools.cache
def _deg_call():
    return pl.kernel(
        _deg_body,
        out_type=jax.ShapeDtypeStruct((NC * NPAD,), jnp.float32),
        mesh=_mesh(),
        scratch_types=(
            [pltpu.VMEM((GSZ,), jnp.int32)] * NB
            + [pltpu.VMEM((GSZ,), jnp.float32),
               pltpu.VMEM_SHARED((NPAD,), jnp.float32),
               pltpu.SemaphoreType.DMA]
        ),
    )


def _sct_body(src_hbm, dst_hbm, y0_hbm, y1_hbm, out0_hbm, out1_hbm, *scr):
    sidxs = scr[0:NB]
    didxs = scr[NB:2 * NB]
    vals0 = scr[2 * NB:3 * NB]
    vals1 = scr[3 * NB:4 * NB]
    gsems = scr[4 * NB:5 * NB]
    sh_y0, sh_y1, sh_s0, sh_s1, ssa, ssb = scr[5 * NB:]
    c = lax.axis_index("c")
    t = lax.axis_index("s")
    gc = jnp.where(c == 0, G_A, G_B)
    base = (c * NSUB * G_A + t * gc) * GSZ
    lo = t * NSLICE

    zb = vals0[0]

    def zfill(k, carry):
        zb[pl.ds(k * 16, 16)] = jnp.zeros((16,), jnp.float32)
        return carry

    lax.fori_loop(0, GSZ // 16, zfill, 0)
    for shs in (sh_s0, sh_s1):
        def zslice(k, carry, shs=shs):
            pltpu.sync_copy(zb, shs.at[pl.ds(lo + k * GSZ, GSZ)])
            return carry

        lax.fori_loop(0, NSLICE // GSZ, zslice, 0)
        pltpu.sync_copy(zb.at[pl.ds(0, NSLICE % GSZ)],
                        shs.at[pl.ds(lo + (NSLICE // GSZ) * GSZ,
                                     NSLICE % GSZ)])
    pltpu.sync_copy(y0_hbm.at[pl.ds(lo, NSLICE)], sh_y0.at[pl.ds(lo, NSLICE)])
    pltpu.sync_copy(y1_hbm.at[pl.ds(lo, NSLICE)], sh_y1.at[pl.ds(lo, NSLICE)])
    plsc.subcore_barrier()

    def sat(g):
        return src_hbm.at[pl.ds(base + g * GSZ, GSZ)]

    def dat(g):
        return dst_hbm.at[pl.ds(base + g * GSZ, GSZ)]

    def body(i, carry):
        g0 = i * NB
        gd = []
        for j in range(NB):
            pltpu.sync_copy(sat(g0 + j), sidxs[j])
            pltpu.sync_copy(dat(g0 + j), didxs[j])
            gd.append((pltpu.async_copy(sh_y0.at[sidxs[j]], vals0[j],
                                        gsems[j]),
                       pltpu.async_copy(sh_y1.at[sidxs[j]], vals1[j],
                                        gsems[j])))
        sd = []
        for j in range(NB):
            gd[j][0].wait()
            gd[j][1].wait()
            sd.append((pltpu.async_copy(vals0[j], sh_s0.at[didxs[j]], ssa,
                                        add=True),
                       pltpu.async_copy(vals1[j], sh_s1.at[didxs[j]], ssb,
                                        add=True)))
        for j in range(NB):
            sd[j][0].wait()
            sd[j][1].wait()
        return carry

    lax.fori_loop(0, gc // NB, body, 0)
    plsc.subcore_barrier()
    pltpu.sync_copy(sh_s0.at[pl.ds(lo, NSLICE)],
                    out0_hbm.at[pl.ds(c * NPAD + lo, NSLICE)])
    pltpu.sync_copy(sh_s1.at[pl.ds(lo, NSLICE)],
                    out1_hbm.at[pl.ds(c * NPAD + lo, NSLICE)])


@functools.cache
def _sct_call():
    return pl.kernel(
        _sct_body,
        out_type=(
            jax.ShapeDtypeStruct((NC * NPAD,), jnp.float32),
            jax.ShapeDtypeStruct((NC * NPAD,), jnp.float32),
        ),
        mesh=_mesh(),
        scratch_types=(
            [pltpu.VMEM((GSZ,), jnp.int32)] * (2 * NB)
            + [pltpu.VMEM((GSZ,), jnp.float32)] * (2 * NB)
            + [pltpu.SemaphoreType.DMA] * NB
            + [pltpu.VMEM_SHARED((NPAD,), jnp.float32)] * 4
            + [pltpu.SemaphoreType.DMA] * 2
        ),
    )


def _prep_body(degp_ref, xT_ref, dinv_ref, yT_ref):
    deg = degp_ref[0] + degp_ref[1] + 1.0
    dinv = lax.rsqrt(deg)
    dinv_ref[...] = dinv
    yT_ref[0] = xT_ref[0] * dinv
    yT_ref[1] = xT_ref[1] * dinv


def _prep_call(degp3, xT3):
    return pl.pallas_call(
        _prep_body,
        out_shape=(
            jax.ShapeDtypeStruct((NB128, 128), jnp.float32),
            jax.ShapeDtypeStruct((IN_DIM, NB128, 128), jnp.float32),
        ),
    )(degp3, xT3)


D1R = 8
D1G = NB128 // D1R


def _d1_body(o00, o01, y0b, o10, o11, y1b, dv, WTr, bc, gc, betac,
             acc0_ref, acc1_ref, scale_ref, shift_ref, ssum, ssq):
    i = pl.program_id(0)

    @pl.when(i == 0)
    def _():
        ssum[...] = jnp.zeros_like(ssum)
        ssq[...] = jnp.zeros_like(ssq)

    acc0 = (o00[...] + o01[...] + y0b[...]) * dv[...]
    acc1 = (o10[...] + o11[...] + y1b[...]) * dv[...]
    acc0_ref[...] = acc0
    acc1_ref[...] = acc1
    w0 = WTr[:, 0:1]
    w1 = WTr[:, 1:2]
    lane = lax.broadcasted_iota(jnp.int32, (1, 128), 1)
    su = ssum[...]
    sq = ssq[...]
    for j in range(D1R):
        h1 = jnp.maximum(w0 * acc0[j:j + 1, :] + w1 * acc1[j:j + 1, :]
                         + bc[...], 0.0)
        node = (i * D1R + j) * 128 + lane
        h1 = jnp.where(node < N, h1, 0.0)
        su = su + h1
        sq = sq + h1 * h1
    ssum[...] = su
    ssq[...] = sq

    @pl.when(i == pl.num_programs(0) - 1)
    def _():
        mean = jnp.sum(su, axis=1, keepdims=True) * (1.0 / N)
        ex2 = jnp.sum(sq, axis=1, keepdims=True) * (1.0 / N)
        var = ex2 - mean * mean
        rstd = lax.rsqrt(var + 1e-5)
        scale_ref[...] = gc[...] * rstd
        shift_ref[...] = betac[...] - mean * gc[...] * rstd


def _d1_call(o00, o01, y0b, o10, o11, y1b, dinvb, WT, bc, gc, betac):
    blk = pl.BlockSpec((D1R, 128), lambda i: (i, 0))
    col = pl.BlockSpec((OUT_DIM, 1), lambda i: (0, 0))
    return pl.pallas_call(
        _d1_body,
        grid=(D1G,),
        in_specs=[blk, blk, blk, blk, blk, blk, blk,
                  pl.BlockSpec((OUT_DIM, IN_DIM), lambda i: (0, 0)),
                  col, col, col],
        out_specs=(blk, blk, col, col),
        out_shape=(
            jax.ShapeDtypeStruct((NB128, 128), jnp.float32),
            jax.ShapeDtypeStruct((NB128, 128), jnp.float32),
            jax.ShapeDtypeStruct((OUT_DIM, 1), jnp.float32),
            jax.ShapeDtypeStruct((OUT_DIM, 1), jnp.float32),
        ),
        scratch_shapes=[
            pltpu.VMEM((OUT_DIM, 128), jnp.float32),
            pltpu.VMEM((OUT_DIM, 128), jnp.float32),
        ],
    )(o00, o01, y0b, o10, o11, y1b, dinvb, WT, bc, gc, betac)


D2R = 8
D2G = NB128 // D2R


def _d2_body(acc0b, acc1b, WTr, bc, scale, shift, fcW_r, fcb_r,
             out1_ref, out2_ref):
    w0 = WTr[:, 0:1]
    w1 = WTr[:, 1:2]
    sc = scale[...]
    sh = shift[...]
    ones = jnp.ones((OUT_DIM, 1), jnp.float32)
    for j in range(D2R):
        h1 = jnp.maximum(w0 * acc0b[j:j + 1, :] + w1 * acc1b[j:j + 1, :]
                         + bc[...], 0.0)
        hbnT = h1 * sc + sh
        h2 = lax.dot_general(hbnT.astype(jnp.bfloat16), fcW_r[...],
                             (((0,), (0,)), ((), ())),
                             preferred_element_type=jnp.float32)
        h2 = h2 + fcb_r[...]
        s = lax.dot_general(jnp.exp(h2), ones,
                            (((1,), (0,)), ((), ())),
                            preferred_element_type=jnp.float32)
        lse = jnp.log(s)
        out1_ref[pl.ds(j * 128, 128), :] = h2 - lse
        out2_ref[pl.ds(j * 128, 128), :] = h2


def _d2_call(acc0, acc1, WT, bc, scale, shift, fcW, fcb2):
    blk = pl.BlockSpec((D2R, 128), lambda i: (i, 0))
    col = pl.BlockSpec((OUT_DIM, 1), lambda i: (0, 0))
    return pl.pallas_call(
        _d2_body,
        grid=(D2G,),
        in_specs=[blk, blk,
                  pl.BlockSpec((OUT_DIM, IN_DIM), lambda i: (0, 0)),
                  col, col, col,
                  pl.BlockSpec((OUT_DIM, OUT_DIM), lambda i: (0, 0)),
                  pl.BlockSpec((1, OUT_DIM), lambda i: (0, 0))],
        out_specs=(
            pl.BlockSpec((D2R * 128, OUT_DIM), lambda i: (i, 0)),
            pl.BlockSpec((D2R * 128, OUT_DIM), lambda i: (i, 0)),
        ),
        out_shape=(
            jax.ShapeDtypeStruct((N, OUT_DIM), jnp.float32),
            jax.ShapeDtypeStruct((N, OUT_DIM), jnp.float32),
        ),
    )(acc0, acc1, WT, bc, scale, shift, fcW, fcb2)


def kernel(x, edge_index, W, b, gamma, beta, fcW, fcb):
    ei = edge_index.astype(jnp.int32)
    padlen = EPAD - E
    pad = jnp.full((padlen,), TRASH, jnp.int32)
    src3 = jnp.concatenate([ei[0], pad])
    dst3 = jnp.concatenate([ei[1], pad])

    xpad = jnp.pad(x, ((0, NPAD - N), (0, 0)))
    xT3 = xpad.T.reshape(IN_DIM, NB128, 128)
    zeros_n = jnp.zeros((NPAD,), jnp.float32)

    degp = _deg_call()(dst3, zeros_n).reshape(NC, NB128, 128)
    dinvb, yT = _prep_call(degp, xT3)
    y0 = yT[0].reshape(NPAD)
    y1 = yT[1].reshape(NPAD)

    s0f, s1f = _sct_call()(src3, dst3, y0, y1)
    s0p = s0f.reshape(NC, NB128, 128)
    s1p = s1f.reshape(NC, NB128, 128)

    WT = W.T
    bc = b.reshape(OUT_DIM, 1)
    gc = gamma.reshape(OUT_DIM, 1)
    betac = beta.reshape(OUT_DIM, 1)
    fcb2 = fcb.reshape(1, OUT_DIM)
    acc0, acc1, scale, shift = _d1_call(
        s0p[0], s0p[1], yT[0], s1p[0], s1p[1], yT[1],
        dinvb, WT, bc, gc, betac)
    out1, out2 = _d2_call(acc0, acc1, WT, bc, scale, shift,
                          fcW.astype(jnp.bfloat16), fcb2)
    return out1, out2
